# Initial kernel scaffold; baseline (speedup 1.0000x reference)
import jax
import jax.numpy as jnp
from jax.experimental import pallas as pl


def kernel(x, edge_index, W1, b1, W2, b2):
    raise NotImplementedError("write your pallas kernel here")


# speedup vs baseline: 18.2618x; 18.2618x over previous
import functools

import jax
import jax.numpy as jnp
from jax import lax
from jax.experimental import pallas as pl
from jax.experimental.pallas import tpu as pltpu
from jax.experimental.pallas import tpu_sc as plsc

N_NODES = 10000
N_EDGES = 160000
NFEAT = 256
H1 = 32
H2 = 128

NC = 2
NS = 16
NW = NC * NS
L = 16

CH = 128
E_PAD = 163840
EPW = E_PAD // NW
NCH = EPW // CH
ACC_R = 10240
STRIPE = ACC_R // NS
DUMMY = N_NODES
NBUF = 8

_mesh = plsc.VectorSubcoreMesh(core_axis_name="c", subcore_axis_name="s")


@functools.partial(
    pl.kernel,
    out_type=jax.ShapeDtypeStruct((NC, ACC_R), jnp.float32),
    mesh=_mesh,
    scratch_types=[
        pltpu.VMEM((NCH, CH), jnp.int32),
        pltpu.VMEM((CH,), jnp.float32),
        pltpu.VMEM((STRIPE,), jnp.float32),
        pltpu.VMEM_SHARED((ACC_R,), jnp.float32),
        pltpu.SemaphoreType.DMA,
    ],
    compiler_params=pltpu.CompilerParams(use_tc_tiling_on_sc=False),
)
def _deg_kernel(dst_hbm, out_hbm, dst_v, ones_v, z_v, acc_sh, sem):
    cid = lax.axis_index("c")
    sid = lax.axis_index("s")
    wid = sid * NC + cid

    def zf(i, _):
        z_v[pl.ds(i * L, L)] = jnp.zeros((L,), jnp.float32)
        return 0

    lax.fori_loop(0, STRIPE // L, zf, 0)

    def of(i, _):
        ones_v[pl.ds(i * L, L)] = jnp.ones((L,), jnp.float32)
        return 0

    lax.fori_loop(0, CH // L, of, 0)

    pltpu.sync_copy(z_v, acc_sh.at[pl.ds(sid * STRIPE, STRIPE)])
    plsc.subcore_barrier()

    pltpu.sync_copy(dst_hbm.at[wid], dst_v)

    def body(j, _):
        pltpu.sync_copy(ones_v, acc_sh.at[dst_v.at[j]], add=True)
        return 0

    lax.fori_loop(0, NCH, body, 0)
    plsc.subcore_barrier()

    pltpu.sync_copy(
        acc_sh.at[pl.ds(sid * STRIPE, STRIPE)],
        out_hbm.at[cid, pl.ds(sid * STRIPE, STRIPE)],
    )


@functools.partial(
    pl.kernel,
    out_type=jax.ShapeDtypeStruct((NC, ACC_R, H1), jnp.float32),
    mesh=_mesh,
    scratch_types=[
        pltpu.VMEM((NCH, CH), jnp.int32),
        pltpu.VMEM((NCH, CH), jnp.int32),
        pltpu.VMEM((NBUF, CH, H1), jnp.float32),
        pltpu.VMEM_SHARED((ACC_R, H1), jnp.float32),
        pltpu.SemaphoreType.DMA,
        pltpu.SemaphoreType.DMA,
    ],
    compiler_params=pltpu.CompilerParams(use_tc_tiling_on_sc=False),
)
def _agg_kernel(y_hbm, src_hbm, dst_hbm, out_hbm, src_v, dst_v, rows_v, acc_sh,
                gsem, ssem):
    cid = lax.axis_index("c")
    sid = lax.axis_index("s")
    wid = sid * NC + cid

    def zf(i, _):
        rows_v[0, i, pl.ds(0, L)] = jnp.zeros((L,), jnp.float32)
        rows_v[0, i, pl.ds(L, L)] = jnp.zeros((L,), jnp.float32)
        return 0

    lax.fori_loop(0, CH, zf, 0)
    for i in range(STRIPE // CH):
        pltpu.sync_copy(rows_v.at[0], acc_sh.at[pl.ds(sid * STRIPE + i * CH, CH)])
    plsc.subcore_barrier()

    pltpu.sync_copy(src_hbm.at[wid], src_v)
    pltpu.sync_copy(dst_hbm.at[wid], dst_v)

    def group(g, _):
        base = g * NBUF
        handles = []
        for b in range(NBUF):
            handles.append(
                pltpu.async_copy(y_hbm.at[src_v.at[base + b]], rows_v.at[b], gsem)
            )
        for h in handles:
            h.wait()
        shandles = []
        for b in range(NBUF):
            shandles.append(
                pltpu.async_copy(rows_v.at[b], acc_sh.at[dst_v.at[base + b]],
                                 ssem, add=True)
            )
        for h in shandles:
            h.wait()
        return 0

    lax.fori_loop(0, NCH // NBUF, group, 0)
    plsc.subcore_barrier()

    pltpu.sync_copy(
        acc_sh.at[pl.ds(sid * STRIPE, STRIPE)],
        out_hbm.at[cid, pl.ds(sid * STRIPE, STRIPE)],
    )


def _tc_a_body(x_ref, w1_ref, b1_ref, degp_ref, y1_ref, selfb_ref, dinv_ref):
    xw = jnp.dot(x_ref[...], w1_ref[...], preferred_element_type=jnp.float32)
    deg = degp_ref[0, :N_NODES, :] + degp_ref[1, :N_NODES, :] + 1.0
    dinv = lax.rsqrt(deg)
    y1_ref[...] = xw * dinv
    selfb_ref[...] = xw / deg + b1_ref[...]
    dinv_ref[...] = dinv


def _tc_b_body(accp_ref, selfb_ref, dinv_ref, y2_ref, self2_ref):
    a = accp_ref[0, :N_NODES, :] + accp_ref[1, :N_NODES, :]
    dinv = dinv_ref[...]
    h = jnp.maximum(a * dinv + selfb_ref[...], 0.0)
    y2_ref[...] = h * dinv
    self2_ref[...] = h * (dinv * dinv)


def _tc_c_body(accp_ref, self2_ref, dinv_ref, w2_ref, b2_ref, out_ref):
    a = accp_ref[0, :N_NODES, :] + accp_ref[1, :N_NODES, :]
    z = a * dinv_ref[...] + self2_ref[...]
    out_ref[...] = (
        jnp.dot(z, w2_ref[...], preferred_element_type=jnp.float32) + b2_ref[...]
    )


_f32 = jnp.float32

_tc_a = pl.pallas_call(
    _tc_a_body,
    out_shape=(
        jax.ShapeDtypeStruct((N_NODES, H1), _f32),
        jax.ShapeDtypeStruct((N_NODES, H1), _f32),
        jax.ShapeDtypeStruct((N_NODES, 1), _f32),
    ),
)

_tc_b = pl.pallas_call(
    _tc_b_body,
    out_shape=(
        jax.ShapeDtypeStruct((N_NODES, H1), _f32),
        jax.ShapeDtypeStruct((N_NODES, H1), _f32),
    ),
)

_tc_c = pl.pallas_call(
    _tc_c_body,
    out_shape=jax.ShapeDtypeStruct((N_NODES, H2), _f32),
)


def kernel(x, edge_index, W1, b1, W2, b2):
    src = edge_index[0].astype(jnp.int32)
    dst = edge_index[1].astype(jnp.int32)
    pad = E_PAD - N_EDGES
    src_p = jnp.concatenate([src, jnp.zeros((pad,), jnp.int32)]).reshape(NW, NCH, CH)
    dst_p = jnp.concatenate([dst, jnp.full((pad,), DUMMY, jnp.int32)]).reshape(
        NW, NCH, CH
    )

    degp = _deg_kernel(dst_p)
    degp3 = degp.reshape(NC, ACC_R, 1)

    y1, selfb, dinv = _tc_a(x, W1, b1.reshape(1, H1), degp3)
    acc1 = _agg_kernel(y1, src_p, dst_p)
    y2, self2 = _tc_b(acc1, selfb, dinv)
    acc2 = _agg_kernel(y2, src_p, dst_p)
    out = _tc_c(acc2, self2, dinv, W2, b2.reshape(1, H2))
    return out


# speedup vs baseline: 30.5175x; 1.6711x over previous
import functools

import jax
import jax.numpy as jnp
from jax import lax
from jax.experimental import pallas as pl
from jax.experimental.pallas import tpu as pltpu
from jax.experimental.pallas import tpu_sc as plsc

N_NODES = 10000
N_EDGES = 160000
NFEAT = 256
H1 = 32
H2 = 128

NC = 2
NS = 16
NW = NC * NS
L = 16

CH = 128
E_PAD = 163840
EPW = E_PAD // NW
NCH = EPW // CH
ACC_R = 10240
STRIPE = ACC_R // NS
DUMMY = N_NODES
NBUF = 8

_mesh = plsc.VectorSubcoreMesh(core_axis_name="c", subcore_axis_name="s")


@functools.partial(
    pl.kernel,
    out_type=jax.ShapeDtypeStruct((NC, ACC_R), jnp.float32),
    mesh=_mesh,
    scratch_types=[
        pltpu.VMEM((NCH, CH), jnp.int32),
        pltpu.VMEM((CH,), jnp.float32),
        pltpu.VMEM((STRIPE,), jnp.float32),
        pltpu.VMEM_SHARED((ACC_R,), jnp.float32),
        pltpu.SemaphoreType.DMA,
    ],
    compiler_params=pltpu.CompilerParams(use_tc_tiling_on_sc=False),
)
def _deg_kernel(dst_hbm, out_hbm, dst_v, ones_v, z_v, acc_sh, sem):
    cid = lax.axis_index("c")
    sid = lax.axis_index("s")
    wid = sid * NC + cid

    def zf(i, _):
        z_v[pl.ds(i * L, L)] = jnp.zeros((L,), jnp.float32)
        return 0

    lax.fori_loop(0, STRIPE // L, zf, 0)

    def of(i, _):
        ones_v[pl.ds(i * L, L)] = jnp.ones((L,), jnp.float32)
        return 0

    lax.fori_loop(0, CH // L, of, 0)

    pltpu.sync_copy(z_v, acc_sh.at[pl.ds(sid * STRIPE, STRIPE)])
    plsc.subcore_barrier()

    pltpu.sync_copy(dst_hbm.at[wid], dst_v)

    def body(j, _):
        pltpu.sync_copy(ones_v, acc_sh.at[dst_v.at[j]], add=True)
        return 0

    lax.fori_loop(0, NCH, body, 0)
    plsc.subcore_barrier()

    pltpu.sync_copy(
        acc_sh.at[pl.ds(sid * STRIPE, STRIPE)],
        out_hbm.at[cid, pl.ds(sid * STRIPE, STRIPE)],
    )


@functools.partial(
    pl.kernel,
    out_type=jax.ShapeDtypeStruct((NC, ACC_R, H1), jnp.float32),
    mesh=_mesh,
    scratch_types=[
        pltpu.VMEM((NCH, CH), jnp.int32),
        pltpu.VMEM((NCH, CH), jnp.int32),
        pltpu.VMEM((NBUF, CH, H1), jnp.float32),
        pltpu.VMEM_SHARED((ACC_R, H1), jnp.float32),
        pltpu.VMEM_SHARED((N_NODES, H1), jnp.float32),
        pltpu.SemaphoreType.DMA,
        pltpu.SemaphoreType.DMA,
    ],
    compiler_params=pltpu.CompilerParams(use_tc_tiling_on_sc=False),
)
def _agg_kernel(y_hbm, src_hbm, dst_hbm, out_hbm, src_v, dst_v, rows_v, acc_sh,
                ytab_sh, gsem, ssem):
    cid = lax.axis_index("c")
    sid = lax.axis_index("s")
    wid = sid * NC + cid

    ytc = pltpu.async_copy(
        y_hbm.at[pl.ds(sid * (N_NODES // NS), N_NODES // NS)],
        ytab_sh.at[pl.ds(sid * (N_NODES // NS), N_NODES // NS)],
        gsem,
    )

    def zf(i, _):
        rows_v[0, i, pl.ds(0, L)] = jnp.zeros((L,), jnp.float32)
        rows_v[0, i, pl.ds(L, L)] = jnp.zeros((L,), jnp.float32)
        return 0

    lax.fori_loop(0, CH, zf, 0)
    for i in range(STRIPE // CH):
        pltpu.sync_copy(rows_v.at[0], acc_sh.at[pl.ds(sid * STRIPE + i * CH, CH)])

    pltpu.sync_copy(src_hbm.at[wid], src_v)
    pltpu.sync_copy(dst_hbm.at[wid], dst_v)
    ytc.wait()
    plsc.subcore_barrier()

    def group(g, _):
        base = g * NBUF
        handles = []
        for b in range(NBUF):
            handles.append(
                pltpu.async_copy(ytab_sh.at[src_v.at[base + b]], rows_v.at[b],
                                 gsem)
            )
        for h in handles:
            h.wait()
        shandles = []
        for b in range(NBUF):
            shandles.append(
                pltpu.async_copy(rows_v.at[b], acc_sh.at[dst_v.at[base + b]],
                                 ssem, add=True)
            )
        for h in shandles:
            h.wait()
        return 0

    lax.fori_loop(0, NCH // NBUF, group, 0)
    plsc.subcore_barrier()

    pltpu.sync_copy(
        acc_sh.at[pl.ds(sid * STRIPE, STRIPE)],
        out_hbm.at[cid, pl.ds(sid * STRIPE, STRIPE)],
    )


def _tc_a_body(x_ref, w1_ref, b1_ref, degp_ref, y1_ref, selfb_ref, dinv_ref):
    xw = jnp.dot(x_ref[...], w1_ref[...], preferred_element_type=jnp.float32)
    deg = degp_ref[0, :N_NODES, :] + degp_ref[1, :N_NODES, :] + 1.0
    dinv = lax.rsqrt(deg)
    y1_ref[...] = xw * dinv
    selfb_ref[...] = xw / deg + b1_ref[...]
    dinv_ref[...] = dinv


def _tc_b_body(accp_ref, selfb_ref, dinv_ref, y2_ref, self2_ref):
    a = accp_ref[0, :N_NODES, :] + accp_ref[1, :N_NODES, :]
    dinv = dinv_ref[...]
    h = jnp.maximum(a * dinv + selfb_ref[...], 0.0)
    y2_ref[...] = h * dinv
    self2_ref[...] = h * (dinv * dinv)


def _tc_c_body(accp_ref, self2_ref, dinv_ref, w2_ref, b2_ref, out_ref):
    a = accp_ref[0, :N_NODES, :] + accp_ref[1, :N_NODES, :]
    z = a * dinv_ref[...] + self2_ref[...]
    out_ref[...] = (
        jnp.dot(z, w2_ref[...], preferred_element_type=jnp.float32) + b2_ref[...]
    )


_f32 = jnp.float32

_tc_a = pl.pallas_call(
    _tc_a_body,
    out_shape=(
        jax.ShapeDtypeStruct((N_NODES, H1), _f32),
        jax.ShapeDtypeStruct((N_NODES, H1), _f32),
        jax.ShapeDtypeStruct((N_NODES, 1), _f32),
    ),
)

_tc_b = pl.pallas_call(
    _tc_b_body,
    out_shape=(
        jax.ShapeDtypeStruct((N_NODES, H1), _f32),
        jax.ShapeDtypeStruct((N_NODES, H1), _f32),
    ),
)

_tc_c = pl.pallas_call(
    _tc_c_body,
    out_shape=jax.ShapeDtypeStruct((N_NODES, H2), _f32),
)


def kernel(x, edge_index, W1, b1, W2, b2):
    src = edge_index[0].astype(jnp.int32)
    dst = edge_index[1].astype(jnp.int32)
    pad = E_PAD - N_EDGES
    src_p = jnp.concatenate([src, jnp.zeros((pad,), jnp.int32)]).reshape(NW, NCH, CH)
    dst_p = jnp.concatenate([dst, jnp.full((pad,), DUMMY, jnp.int32)]).reshape(
        NW, NCH, CH
    )

    degp = _deg_kernel(dst_p)
    degp3 = degp.reshape(NC, ACC_R, 1)

    y1, selfb, dinv = _tc_a(x, W1, b1.reshape(1, H1), degp3)
    acc1 = _agg_kernel(y1, src_p, dst_p)
    y2, self2 = _tc_b(acc1, selfb, dinv)
    acc2 = _agg_kernel(y2, src_p, dst_p)
    out = _tc_c(acc2, self2, dinv, W2, b2.reshape(1, H2))
    return out


# speedup vs baseline: 33.4685x; 1.0967x over previous
import functools

import jax
import jax.numpy as jnp
from jax import lax
from jax.experimental import pallas as pl
from jax.experimental.pallas import tpu as pltpu
from jax.experimental.pallas import tpu_sc as plsc

N_NODES = 10000
N_EDGES = 160000
NFEAT = 256
H1 = 32
H2 = 128

NC = 2
NS = 16
NW = NC * NS
L = 16

CH = 128
E_PAD = 163840
EPW = E_PAD // NW
NCH = EPW // CH
GRP = 4
NGRP = NCH // GRP
ACC_R = 10240
STRIPE = ACC_R // NS
YSTRIPE = N_NODES // NS
DUMMY = N_NODES

_mesh = plsc.VectorSubcoreMesh(core_axis_name="c", subcore_axis_name="s")
_scp = pltpu.CompilerParams(use_tc_tiling_on_sc=False)


@functools.partial(
    pl.kernel,
    out_type=jax.ShapeDtypeStruct((NC, ACC_R), jnp.float32),
    mesh=_mesh,
    scratch_types=[
        pltpu.VMEM((NCH, CH), jnp.int32),
        pltpu.VMEM((CH,), jnp.float32),
        pltpu.VMEM((STRIPE,), jnp.float32),
        pltpu.VMEM_SHARED((ACC_R,), jnp.float32),
        pltpu.SemaphoreType.DMA,
    ],
    compiler_params=_scp,
)
def _deg_kernel(dst_hbm, out_hbm, dst_v, ones_v, z_v, acc_sh, sem):
    cid = lax.axis_index("c")
    sid = lax.axis_index("s")
    wid = sid * NC + cid

    def zf(i, _):
        z_v[pl.ds(i * L, L)] = jnp.zeros((L,), jnp.float32)
        return 0

    lax.fori_loop(0, STRIPE // L, zf, 0)

    def of(i, _):
        ones_v[pl.ds(i * L, L)] = jnp.ones((L,), jnp.float32)
        return 0

    lax.fori_loop(0, CH // L, of, 0)

    pltpu.sync_copy(z_v, acc_sh.at[pl.ds(sid * STRIPE, STRIPE)])
    plsc.subcore_barrier()

    pltpu.sync_copy(dst_hbm.at[wid], dst_v)

    handles = [
        pltpu.async_copy(ones_v, acc_sh.at[dst_v.at[j]], sem, add=True)
        for j in range(NCH)
    ]
    for h in handles:
        h.wait()
    plsc.subcore_barrier()

    pltpu.sync_copy(
        acc_sh.at[pl.ds(sid * STRIPE, STRIPE)],
        out_hbm.at[cid, pl.ds(sid * STRIPE, STRIPE)],
    )


@functools.partial(
    pl.kernel,
    out_type=jax.ShapeDtypeStruct((NC, ACC_R, H1), jnp.float32),
    mesh=_mesh,
    scratch_types=[
        pltpu.VMEM((NCH, CH), jnp.int32),
        pltpu.VMEM((NCH, CH), jnp.int32),
        pltpu.VMEM((3 * GRP, CH, H1), jnp.float32),
        pltpu.VMEM_SHARED((ACC_R, H1), jnp.float32),
        pltpu.VMEM_SHARED((N_NODES, H1), jnp.float32),
        pltpu.SemaphoreType.DMA,
        pltpu.SemaphoreType.DMA,
    ],
    compiler_params=_scp,
)
def _agg_kernel(y_hbm, src_hbm, dst_hbm, out_hbm, src_v, dst_v, rows_v, acc_sh,
                ytab_sh, gsem, ssem):
    cid = lax.axis_index("c")
    sid = lax.axis_index("s")
    wid = sid * NC + cid

    ytc = pltpu.async_copy(
        y_hbm.at[pl.ds(sid * YSTRIPE, YSTRIPE)],
        ytab_sh.at[pl.ds(sid * YSTRIPE, YSTRIPE)],
        gsem,
    )

    def zf(i, _):
        rows_v[0, i, pl.ds(0, L)] = jnp.zeros((L,), jnp.float32)
        rows_v[0, i, pl.ds(L, L)] = jnp.zeros((L,), jnp.float32)
        return 0

    lax.fori_loop(0, CH, zf, 0)
    for i in range(STRIPE // CH):
        pltpu.sync_copy(rows_v.at[0], acc_sh.at[pl.ds(sid * STRIPE + i * CH, CH)])

    pltpu.sync_copy(src_hbm.at[wid], src_v)
    pltpu.sync_copy(dst_hbm.at[wid], dst_v)
    ytc.wait()
    plsc.subcore_barrier()

    def fire_gathers(g):
        s = (g % 3) * GRP
        return [
            pltpu.async_copy(ytab_sh.at[src_v.at[g * GRP + b]],
                             rows_v.at[s + b], gsem)
            for b in range(GRP)
        ]

    def fire_scatters(g):
        s = (g % 3) * GRP
        return [
            pltpu.async_copy(rows_v.at[s + b],
                             acc_sh.at[dst_v.at[g * GRP + b]], ssem, add=True)
            for b in range(GRP)
        ]

    gh = {0: fire_gathers(0), 1: fire_gathers(1)}
    sh = {}
    for g in range(NGRP):
        for h in gh.pop(g):
            h.wait()
        sh[g] = fire_scatters(g)
        if g + 2 < NGRP:
            if g - 1 in sh:
                for h in sh.pop(g - 1):
                    h.wait()
            gh[g + 2] = fire_gathers(g + 2)
    for g in sorted(sh):
        for h in sh[g]:
            h.wait()
    plsc.subcore_barrier()

    pltpu.sync_copy(
        acc_sh.at[pl.ds(sid * STRIPE, STRIPE)],
        out_hbm.at[cid, pl.ds(sid * STRIPE, STRIPE)],
    )


def _tc_a_body(x_ref, w1_ref, b1_ref, degp_ref, y1_ref, selfb_ref, dinvb_ref):
    xw = jnp.dot(x_ref[...], w1_ref[...], preferred_element_type=jnp.float32)
    deg = degp_ref[0, :N_NODES, :] + degp_ref[1, :N_NODES, :] + 1.0
    dinvb = jnp.broadcast_to(lax.rsqrt(deg), (N_NODES, H1))
    y1_ref[...] = xw * dinvb
    selfb_ref[...] = xw * (dinvb * dinvb) + b1_ref[...]
    dinvb_ref[...] = dinvb


def _tc_b_body(accp_ref, selfb_ref, dinvb_ref, y2_ref, self2_ref):
    a = accp_ref[0, :N_NODES, :] + accp_ref[1, :N_NODES, :]
    dinvb = dinvb_ref[...]
    h = jnp.maximum(a * dinvb + selfb_ref[...], 0.0)
    y2_ref[...] = h * dinvb
    self2_ref[...] = h * (dinvb * dinvb)


def _tc_c_body(accp_ref, self2_ref, dinvb_ref, w2_ref, b2_ref, out_ref):
    a = accp_ref[0, :N_NODES, :] + accp_ref[1, :N_NODES, :]
    z = a * dinvb_ref[...] + self2_ref[...]
    out_ref[...] = (
        jnp.dot(z, w2_ref[...], preferred_element_type=jnp.float32) + b2_ref[...]
    )


_f32 = jnp.float32

_tc_a = pl.pallas_call(
    _tc_a_body,
    out_shape=(
        jax.ShapeDtypeStruct((N_NODES, H1), _f32),
        jax.ShapeDtypeStruct((N_NODES, H1), _f32),
        jax.ShapeDtypeStruct((N_NODES, H1), _f32),
    ),
)

_tc_b = pl.pallas_call(
    _tc_b_body,
    out_shape=(
        jax.ShapeDtypeStruct((N_NODES, H1), _f32),
        jax.ShapeDtypeStruct((N_NODES, H1), _f32),
    ),
)

_tc_c = pl.pallas_call(
    _tc_c_body,
    out_shape=jax.ShapeDtypeStruct((N_NODES, H2), _f32),
)


def kernel(x, edge_index, W1, b1, W2, b2):
    ei = edge_index.astype(jnp.int32)
    pad = jnp.broadcast_to(
        jnp.array([[0], [DUMMY]], jnp.int32), (2, E_PAD - N_EDGES)
    )
    ei_p = jnp.concatenate([ei, pad], axis=1)
    src_p = ei_p[0].reshape(NW, NCH, CH)
    dst_p = ei_p[1].reshape(NW, NCH, CH)

    degp = _deg_kernel(dst_p)
    degp3 = degp.reshape(NC, ACC_R, 1)

    y1, selfb, dinvb = _tc_a(x, W1, b1.reshape(1, H1), degp3)
    acc1 = _agg_kernel(y1, src_p, dst_p)
    y2, self2 = _tc_b(acc1, selfb, dinvb)
    acc2 = _agg_kernel(y2, src_p, dst_p)
    out = _tc_c(acc2, self2, dinvb, W2, b2.reshape(1, H2))
    return out


# speedup vs baseline: 38.9353x; 1.1633x over previous
import functools

import jax
import jax.numpy as jnp
from jax import lax
from jax.experimental import pallas as pl
from jax.experimental.pallas import tpu as pltpu
from jax.experimental.pallas import tpu_sc as plsc

N_NODES = 10000
N_EDGES = 160000
NFEAT = 256
H1 = 32
H2 = 128

NC = 2
NS = 16
NW = NC * NS
L = 16

CH = 128
E_PAD = 163840
EPW = E_PAD // NW
NCH = EPW // CH
GRP = 4
NGRP = NCH // GRP
ACC_R = 10240
STRIPE = ACC_R // NS
NCK = STRIPE // CH
DUMMY = N_NODES
AP4 = ACC_R // 4
NP4 = N_NODES // 4

_mesh = plsc.VectorSubcoreMesh(core_axis_name="c", subcore_axis_name="s")
_scp = pltpu.CompilerParams(use_tc_tiling_on_sc=False, needs_layout_passes=False)
_f32 = jnp.float32


def _rsqrt16(x):
    i = plsc.bitcast(x, jnp.int32)
    y = plsc.bitcast(jnp.int32(0x5F3759DF) - (i >> 1), jnp.float32)
    y = y * (1.5 - 0.5 * x * y * y)
    y = y * (1.5 - 0.5 * x * y * y)
    y = y * (1.5 - 0.5 * x * y * y)
    return y


@functools.partial(
    pl.kernel,
    out_type=jax.ShapeDtypeStruct((NC, ACC_R), _f32),
    mesh=_mesh,
    scratch_types=[
        pltpu.VMEM((NCH, CH), jnp.int32),
        pltpu.VMEM((CH,), _f32),
        pltpu.VMEM((STRIPE,), _f32),
        pltpu.VMEM_SHARED((ACC_R,), _f32),
        pltpu.SemaphoreType.DMA,
    ],
    compiler_params=_scp,
)
def _deg_kernel(dst_hbm, out_hbm, dst_v, ones_v, z_v, acc_sh, sem):
    cid = lax.axis_index("c")
    sid = lax.axis_index("s")
    wid = sid * NC + cid

    def zf(i, _):
        z_v[pl.ds(i * L, L)] = jnp.zeros((L,), _f32)
        return 0

    lax.fori_loop(0, STRIPE // L, zf, 0)

    def of(i, _):
        ones_v[pl.ds(i * L, L)] = jnp.ones((L,), _f32)
        return 0

    lax.fori_loop(0, CH // L, of, 0)

    pltpu.sync_copy(z_v, acc_sh.at[pl.ds(sid * STRIPE, STRIPE)])
    plsc.subcore_barrier()

    pltpu.sync_copy(dst_hbm.at[wid], dst_v)

    handles = [
        pltpu.async_copy(ones_v, acc_sh.at[dst_v.at[j]], sem, add=True)
        for j in range(NCH)
    ]
    for h in handles:
        h.wait()
    plsc.subcore_barrier()

    pltpu.sync_copy(
        acc_sh.at[pl.ds(sid * STRIPE, STRIPE)],
        out_hbm.at[cid, pl.ds(sid * STRIPE, STRIPE)],
    )


def _stage_dinv(degp_hbm, deg0_v, dinv_v, base):
    pltpu.sync_copy(degp_hbm.at[0, pl.ds(base, STRIPE)], deg0_v)
    pltpu.sync_copy(degp_hbm.at[1, pl.ds(base, STRIPE)], dinv_v)

    def body(i, _):
        d = deg0_v[pl.ds(i * L, L)] + dinv_v[pl.ds(i * L, L)] + 1.0
        dinv_v[pl.ds(i * L, L)] = _rsqrt16(d)
        return 0

    lax.fori_loop(0, STRIPE // L, body, 0)


def _zero_acc(rows_v, acc_sh, sid):
    def zf(i, _):
        rows_v[0, i, pl.ds(0, L)] = jnp.zeros((L,), _f32)
        rows_v[0, i, pl.ds(L, L)] = jnp.zeros((L,), _f32)
        return 0

    lax.fori_loop(0, CH, zf, 0)
    for i in range(NCK):
        pltpu.sync_copy(rows_v.at[0], acc_sh.at[pl.ds(sid * STRIPE + i * CH, CH)])


def _agg_pipeline(src_v, dst_v, rows_v, acc_sh, ytab_sh, gsem, ssem):
    def fire_gathers(g):
        s = (g % 3) * GRP
        return [
            pltpu.async_copy(ytab_sh.at[src_v.at[g * GRP + b]],
                             rows_v.at[s + b], gsem)
            for b in range(GRP)
        ]

    def fire_scatters(g):
        s = (g % 3) * GRP
        return [
            pltpu.async_copy(rows_v.at[s + b],
                             acc_sh.at[dst_v.at[g * GRP + b]], ssem, add=True)
            for b in range(GRP)
        ]

    gh = {0: fire_gathers(0), 1: fire_gathers(1)}
    sh = {}
    for g in range(NGRP):
        for h in gh.pop(g):
            h.wait()
        sh[g] = fire_scatters(g)
        if g + 2 < NGRP:
            if g - 1 in sh:
                for h in sh.pop(g - 1):
                    h.wait()
            gh[g + 2] = fire_gathers(g + 2)
    for g in sorted(sh):
        for h in sh[g]:
            h.wait()


def _writeout(acc_sh, out_hbm, cid, sid):
    pltpu.sync_copy(
        acc_sh.at[pl.ds(sid * STRIPE, STRIPE)],
        out_hbm.at[cid, pl.ds(sid * STRIPE, STRIPE)],
    )


@functools.partial(
    pl.kernel,
    out_type=jax.ShapeDtypeStruct((NC, ACC_R, H1), _f32),
    mesh=_mesh,
    scratch_types=[
        pltpu.VMEM((NCH, CH), jnp.int32),
        pltpu.VMEM((NCH, CH), jnp.int32),
        pltpu.VMEM((3 * GRP, CH, H1), _f32),
        pltpu.VMEM((STRIPE,), _f32),
        pltpu.VMEM((STRIPE,), _f32),
        pltpu.VMEM_SHARED((ACC_R, H1), _f32),
        pltpu.VMEM_SHARED((ACC_R, H1), _f32),
        pltpu.SemaphoreType.DMA,
        pltpu.SemaphoreType.DMA,
    ],
    compiler_params=_scp,
)
def _agg1_kernel(xw_hbm, degp_hbm, src_hbm, dst_hbm, out_hbm,
                 src_v, dst_v, rows_v, deg0_v, dinv_v, acc_sh, ytab_sh,
                 gsem, ssem):
    cid = lax.axis_index("c")
    sid = lax.axis_index("s")
    wid = sid * NC + cid
    base = sid * STRIPE

    _zero_acc(rows_v, acc_sh, sid)
    _stage_dinv(degp_hbm, deg0_v, dinv_v, base)

    for c in range(NCK):
        pltpu.sync_copy(xw_hbm.at[pl.ds(base + c * CH, CH)], rows_v.at[1])

        def scale(j, _, c=c):
            dvec = dinv_v[pl.ds(c * CH + j * L, L)]
            for i in range(L):
                dv = jnp.full((L,), dvec[i], _f32)
                r = j * L + i
                rows_v[1, r, pl.ds(0, L)] = rows_v[1, r, pl.ds(0, L)] * dv
                rows_v[1, r, pl.ds(L, L)] = rows_v[1, r, pl.ds(L, L)] * dv
            return 0

        lax.fori_loop(0, CH // L, scale, 0)
        pltpu.sync_copy(rows_v.at[1], ytab_sh.at[pl.ds(base + c * CH, CH)])

    pltpu.sync_copy(src_hbm.at[wid], src_v)
    pltpu.sync_copy(dst_hbm.at[wid], dst_v)
    plsc.subcore_barrier()
    _agg_pipeline(src_v, dst_v, rows_v, acc_sh, ytab_sh, gsem, ssem)
    plsc.subcore_barrier()
    _writeout(acc_sh, out_hbm, cid, sid)


@functools.partial(
    pl.kernel,
    out_type=[
        jax.ShapeDtypeStruct((NC, ACC_R, H1), _f32),
        jax.ShapeDtypeStruct((ACC_R, H1), _f32),
        jax.ShapeDtypeStruct((ACC_R, H1), _f32),
    ],
    mesh=_mesh,
    scratch_types=[
        pltpu.VMEM((NCH, CH), jnp.int32),
        pltpu.VMEM((NCH, CH), jnp.int32),
        pltpu.VMEM((3 * GRP, CH, H1), _f32),
        pltpu.VMEM((STRIPE,), _f32),
        pltpu.VMEM((STRIPE,), _f32),
        pltpu.VMEM((H1,), _f32),
        pltpu.VMEM_SHARED((ACC_R, H1), _f32),
        pltpu.VMEM_SHARED((ACC_R, H1), _f32),
        pltpu.SemaphoreType.DMA,
        pltpu.SemaphoreType.DMA,
    ],
    compiler_params=_scp,
)
def _agg2_kernel(acc1_hbm, xw_hbm, degp_hbm, b1_hbm, src_hbm, dst_hbm,
                 out_hbm, self2_hbm, dinve_hbm,
                 src_v, dst_v, rows_v, deg0_v, dinv_v, b_v, acc_sh, ytab_sh,
                 gsem, ssem):
    cid = lax.axis_index("c")
    sid = lax.axis_index("s")
    wid = sid * NC + cid
    base = sid * STRIPE

    _zero_acc(rows_v, acc_sh, sid)
    _stage_dinv(degp_hbm, deg0_v, dinv_v, base)
    pltpu.sync_copy(b1_hbm, b_v)
    b_lo = b_v[pl.ds(0, L)]
    b_hi = b_v[pl.ds(L, L)]

    for c in range(NCK):
        lo = base + c * CH
        pltpu.sync_copy(acc1_hbm.at[0, pl.ds(lo, CH)], rows_v.at[1])
        pltpu.sync_copy(acc1_hbm.at[1, pl.ds(lo, CH)], rows_v.at[2])
        pltpu.sync_copy(xw_hbm.at[pl.ds(lo, CH)], rows_v.at[3])

        def build(j, _, c=c):
            dvec = dinv_v[pl.ds(c * CH + j * L, L)]
            for i in range(L):
                dv = jnp.full((L,), dvec[i], _f32)
                iv = dv * dv
                r = j * L + i
                for off, bb in ((0, b_lo), (L, b_hi)):
                    a = rows_v[1, r, pl.ds(off, L)] + rows_v[2, r, pl.ds(off, L)]
                    xwv = rows_v[3, r, pl.ds(off, L)]
                    hh = jnp.maximum(a * dv + xwv * iv + bb, 0.0)
                    rows_v[3, r, pl.ds(off, L)] = hh * dv
                    rows_v[1, r, pl.ds(off, L)] = hh * iv
                    rows_v[2, r, pl.ds(off, L)] = dv
            return 0

        lax.fori_loop(0, CH // L, build, 0)
        pltpu.sync_copy(rows_v.at[3], ytab_sh.at[pl.ds(lo, CH)])

        @pl.when(cid == 0)
        def _():
            pltpu.sync_copy(rows_v.at[1], self2_hbm.at[pl.ds(lo, CH)])
            pltpu.sync_copy(rows_v.at[2], dinve_hbm.at[pl.ds(lo, CH)])

    pltpu.sync_copy(src_hbm.at[wid], src_v)
    pltpu.sync_copy(dst_hbm.at[wid], dst_v)
    plsc.subcore_barrier()
    _agg_pipeline(src_v, dst_v, rows_v, acc_sh, ytab_sh, gsem, ssem)
    plsc.subcore_barrier()
    _writeout(acc_sh, out_hbm, cid, sid)


def _tc_mm1_body(x_ref, w1_ref, o_ref):
    o_ref[...] = jnp.dot(x_ref[...], w1_ref[...],
                         preferred_element_type=_f32)


_tc_mm1 = pl.pallas_call(
    _tc_mm1_body,
    out_shape=jax.ShapeDtypeStruct((N_NODES, H1), _f32),
)


def _tc_out_body(accp_ref, self2_ref, dinvp_ref, w24_ref, b2t_ref, out_ref):
    a = accp_ref[0, :NP4, :] + accp_ref[1, :NP4, :]
    zp = a * dinvp_ref[:NP4, :] + self2_ref[:NP4, :]
    out_ref[...] = (
        jnp.dot(zp, w24_ref[...], preferred_element_type=_f32) + b2t_ref[...]
    )


_tc_out = pl.pallas_call(
    _tc_out_body,
    out_shape=jax.ShapeDtypeStruct((NP4, 4 * H2), _f32),
)


def kernel(x, edge_index, W1, b1, W2, b2):
    ei = edge_index.astype(jnp.int32)
    pad = jnp.broadcast_to(
        jnp.array([[0], [DUMMY]], jnp.int32), (2, E_PAD - N_EDGES)
    )
    ei_p = jnp.concatenate([ei, pad], axis=1)
    src_p = ei_p[0].reshape(NW, NCH, CH)
    dst_p = ei_p[1].reshape(NW, NCH, CH)

    degp = _deg_kernel(dst_p)
    xw = _tc_mm1(x, W1)
    xw_p = jnp.pad(xw, ((0, ACC_R - N_NODES), (0, 0)))

    acc1 = _agg1_kernel(xw_p, degp, src_p, dst_p)
    acc2, self2, dinve = _agg2_kernel(acc1, xw_p, degp, b1, src_p, dst_p)

    w24 = jax.scipy.linalg.block_diag(W2, W2, W2, W2)
    b2t = jnp.tile(b2, 4).reshape(1, 4 * H2)
    out4 = _tc_out(
        acc2.reshape(NC, AP4, 4 * H1),
        self2.reshape(AP4, 4 * H1),
        dinve.reshape(AP4, 4 * H1),
        w24,
        b2t,
    )
    return out4.reshape(N_NODES, H2)


# speedup vs baseline: 43.4233x; 1.1153x over previous
import functools

import jax
import jax.numpy as jnp
from jax import lax
from jax.experimental import pallas as pl
from jax.experimental.pallas import tpu as pltpu
from jax.experimental.pallas import tpu_sc as plsc

N_NODES = 10000
N_EDGES = 160000
NFEAT = 256
H1 = 32
H2 = 128

NC = 2
NS = 16
NW = NC * NS
L = 16

CH = 128
E_PAD = 163840
EPW = E_PAD // NW
NCH = EPW // CH
GRP = 4
NGRP = NCH // GRP
ACC_R = 10240
STRIPE = ACC_R // NS
NCK = STRIPE // CH
DUMMY = N_NODES
AP4 = ACC_R // 4
NP4 = N_NODES // 4

_mesh = plsc.VectorSubcoreMesh(core_axis_name="c", subcore_axis_name="s")
_scp = pltpu.CompilerParams(use_tc_tiling_on_sc=False, needs_layout_passes=False)
_f32 = jnp.float32


def _rsqrt16(x):
    i = plsc.bitcast(x, jnp.int32)
    y = plsc.bitcast(jnp.int32(0x5F3759DF) - (i >> 1), jnp.float32)
    y = y * (1.5 - 0.5 * x * y * y)
    y = y * (1.5 - 0.5 * x * y * y)
    y = y * (1.5 - 0.5 * x * y * y)
    return y


@functools.partial(
    pl.kernel,
    out_type=jax.ShapeDtypeStruct((NC, ACC_R), _f32),
    mesh=_mesh,
    scratch_types=[
        pltpu.VMEM((NCH, CH), jnp.int32),
        pltpu.VMEM((CH,), _f32),
        pltpu.VMEM((STRIPE,), _f32),
        pltpu.VMEM_SHARED((ACC_R,), _f32),
        pltpu.SemaphoreType.DMA,
    ],
    compiler_params=_scp,
)
def _deg_kernel(dst_hbm, out_hbm, dst_v, ones_v, z_v, acc_sh, sem):
    cid = lax.axis_index("c")
    sid = lax.axis_index("s")
    wid = sid * NC + cid

    def zf(i, _):
        z_v[pl.ds(i * L, L)] = jnp.zeros((L,), _f32)
        return 0

    lax.fori_loop(0, STRIPE // L, zf, 0)

    def of(i, _):
        ones_v[pl.ds(i * L, L)] = jnp.ones((L,), _f32)
        return 0

    lax.fori_loop(0, CH // L, of, 0)

    pltpu.sync_copy(z_v, acc_sh.at[pl.ds(sid * STRIPE, STRIPE)])
    plsc.subcore_barrier()

    pltpu.sync_copy(dst_hbm.at[wid], dst_v)

    handles = [
        pltpu.async_copy(ones_v, acc_sh.at[dst_v.at[j]], sem, add=True)
        for j in range(NCH)
    ]
    for h in handles:
        h.wait()
    plsc.subcore_barrier()

    pltpu.sync_copy(
        acc_sh.at[pl.ds(sid * STRIPE, STRIPE)],
        out_hbm.at[cid, pl.ds(sid * STRIPE, STRIPE)],
    )


def _stage_dinv(degp_hbm, deg0_v, dinv_v, base):
    pltpu.sync_copy(degp_hbm.at[0, pl.ds(base, STRIPE)], deg0_v)
    pltpu.sync_copy(degp_hbm.at[1, pl.ds(base, STRIPE)], dinv_v)

    def body(i, _):
        d = deg0_v[pl.ds(i * L, L)] + dinv_v[pl.ds(i * L, L)] + 1.0
        dinv_v[pl.ds(i * L, L)] = _rsqrt16(d)
        return 0

    lax.fori_loop(0, STRIPE // L, body, 0)


def _zero_acc(rows_v, acc_sh, sid):
    def zf(i, _):
        rows_v[0, i, pl.ds(0, L)] = jnp.zeros((L,), _f32)
        rows_v[0, i, pl.ds(L, L)] = jnp.zeros((L,), _f32)
        return 0

    lax.fori_loop(0, CH, zf, 0)
    for i in range(NCK):
        pltpu.sync_copy(rows_v.at[0], acc_sh.at[pl.ds(sid * STRIPE + i * CH, CH)])


def _agg_pipeline(src_v, dst_v, rows_v, acc_sh, ytab_sh, gsem, ssem):
    def fire_gathers(g):
        s = (g % 3) * GRP
        return [
            pltpu.async_copy(ytab_sh.at[src_v.at[g * GRP + b]],
                             rows_v.at[s + b], gsem)
            for b in range(GRP)
        ]

    def fire_scatters(g):
        s = (g % 3) * GRP
        return [
            pltpu.async_copy(rows_v.at[s + b],
                             acc_sh.at[dst_v.at[g * GRP + b]], ssem, add=True)
            for b in range(GRP)
        ]

    gh = {0: fire_gathers(0), 1: fire_gathers(1)}
    sh = {}
    for g in range(NGRP):
        for h in gh.pop(g):
            h.wait()
        sh[g] = fire_scatters(g)
        if g + 2 < NGRP:
            if g - 1 in sh:
                for h in sh.pop(g - 1):
                    h.wait()
            gh[g + 2] = fire_gathers(g + 2)
    for g in sorted(sh):
        for h in sh[g]:
            h.wait()


def _writeout(acc_sh, out_hbm, cid, sid):
    pltpu.sync_copy(
        acc_sh.at[pl.ds(sid * STRIPE, STRIPE)],
        out_hbm.at[cid, pl.ds(sid * STRIPE, STRIPE)],
    )


@functools.partial(
    pl.kernel,
    out_type=jax.ShapeDtypeStruct((NC, ACC_R, H1), _f32),
    mesh=_mesh,
    scratch_types=[
        pltpu.VMEM((NCH, CH), jnp.int32),
        pltpu.VMEM((NCH, CH), jnp.int32),
        pltpu.VMEM((3 * GRP, CH, H1), _f32),
        pltpu.VMEM((STRIPE,), _f32),
        pltpu.VMEM((STRIPE,), _f32),
        pltpu.VMEM_SHARED((ACC_R, H1), _f32),
        pltpu.VMEM_SHARED((ACC_R, H1), _f32),
        pltpu.SemaphoreType.DMA,
        pltpu.SemaphoreType.DMA,
    ],
    compiler_params=_scp,
)
def _agg1_kernel(xw_hbm, degp_hbm, src_hbm, dst_hbm, out_hbm,
                 src_v, dst_v, rows_v, deg0_v, dinv_v, acc_sh, ytab_sh,
                 gsem, ssem):
    cid = lax.axis_index("c")
    sid = lax.axis_index("s")
    wid = sid * NC + cid
    base = sid * STRIPE

    _zero_acc(rows_v, acc_sh, sid)
    _stage_dinv(degp_hbm, deg0_v, dinv_v, base)

    def _buf(c):
        return 1 + (c % 2)

    stg = {0: pltpu.async_copy(xw_hbm.at[pl.ds(base, CH)], rows_v.at[1], gsem)}
    for c in range(NCK):
        if c + 1 < NCK:
            stg[c + 1] = pltpu.async_copy(
                xw_hbm.at[pl.ds(base + (c + 1) * CH, CH)],
                rows_v.at[_buf(c + 1)], gsem)
        stg.pop(c).wait()
        bi = _buf(c)

        def scale(j, _, c=c, bi=bi):
            dvec = dinv_v[pl.ds(c * CH + j * L, L)]
            for i in range(L):
                dv = jnp.full((L,), dvec[i], _f32)
                r = j * L + i
                rows_v[bi, r, pl.ds(0, L)] = rows_v[bi, r, pl.ds(0, L)] * dv
                rows_v[bi, r, pl.ds(L, L)] = rows_v[bi, r, pl.ds(L, L)] * dv
            return 0

        lax.fori_loop(0, CH // L, scale, 0)
        pltpu.sync_copy(rows_v.at[bi], ytab_sh.at[pl.ds(base + c * CH, CH)])

    pltpu.sync_copy(src_hbm.at[wid], src_v)
    pltpu.sync_copy(dst_hbm.at[wid], dst_v)
    plsc.subcore_barrier()
    _agg_pipeline(src_v, dst_v, rows_v, acc_sh, ytab_sh, gsem, ssem)
    plsc.subcore_barrier()
    _writeout(acc_sh, out_hbm, cid, sid)


@functools.partial(
    pl.kernel,
    out_type=[
        jax.ShapeDtypeStruct((NC, ACC_R, H1), _f32),
        jax.ShapeDtypeStruct((ACC_R, H1), _f32),
        jax.ShapeDtypeStruct((ACC_R, H1), _f32),
    ],
    mesh=_mesh,
    scratch_types=[
        pltpu.VMEM((NCH, CH), jnp.int32),
        pltpu.VMEM((NCH, CH), jnp.int32),
        pltpu.VMEM((3 * GRP, CH, H1), _f32),
        pltpu.VMEM((STRIPE,), _f32),
        pltpu.VMEM((STRIPE,), _f32),
        pltpu.VMEM((H1,), _f32),
        pltpu.VMEM_SHARED((ACC_R, H1), _f32),
        pltpu.VMEM_SHARED((ACC_R, H1), _f32),
        pltpu.SemaphoreType.DMA,
        pltpu.SemaphoreType.DMA,
    ],
    compiler_params=_scp,
)
def _agg2_kernel(acc1_hbm, xw_hbm, degp_hbm, b1_hbm, src_hbm, dst_hbm,
                 out_hbm, self2_hbm, dinve_hbm,
                 src_v, dst_v, rows_v, deg0_v, dinv_v, b_v, acc_sh, ytab_sh,
                 gsem, ssem):
    cid = lax.axis_index("c")
    sid = lax.axis_index("s")
    wid = sid * NC + cid
    base = sid * STRIPE

    _zero_acc(rows_v, acc_sh, sid)
    _stage_dinv(degp_hbm, deg0_v, dinv_v, base)
    pltpu.sync_copy(b1_hbm, b_v)
    b_lo = b_v[pl.ds(0, L)]
    b_hi = b_v[pl.ds(L, L)]

    def _fire_stage(c):
        lo = base + c * CH
        s = 1 + (c % 2) * 3
        return [
            pltpu.async_copy(acc1_hbm.at[0, pl.ds(lo, CH)], rows_v.at[s], gsem),
            pltpu.async_copy(acc1_hbm.at[1, pl.ds(lo, CH)], rows_v.at[s + 1],
                             gsem),
            pltpu.async_copy(xw_hbm.at[pl.ds(lo, CH)], rows_v.at[s + 2], gsem),
        ]

    stg = {0: _fire_stage(0)}
    for c in range(NCK):
        lo = base + c * CH
        s = 1 + (c % 2) * 3
        if c + 1 < NCK:
            stg[c + 1] = _fire_stage(c + 1)
        for h in stg.pop(c):
            h.wait()

        def build(j, _, c=c, s=s):
            dvec = dinv_v[pl.ds(c * CH + j * L, L)]
            for i in range(L):
                dv = jnp.full((L,), dvec[i], _f32)
                iv = dv * dv
                r = j * L + i
                for off, bb in ((0, b_lo), (L, b_hi)):
                    a = (rows_v[s, r, pl.ds(off, L)]
                         + rows_v[s + 1, r, pl.ds(off, L)])
                    xwv = rows_v[s + 2, r, pl.ds(off, L)]
                    hh = jnp.maximum(a * dv + xwv * iv + bb, 0.0)
                    rows_v[s + 2, r, pl.ds(off, L)] = hh * dv
                    rows_v[s, r, pl.ds(off, L)] = hh * iv
                    rows_v[s + 1, r, pl.ds(off, L)] = dv
            return 0

        lax.fori_loop(0, CH // L, build, 0)
        pltpu.sync_copy(rows_v.at[s + 2], ytab_sh.at[pl.ds(lo, CH)])

        @pl.when(cid == 0)
        def _(s=s, lo=lo):
            pltpu.sync_copy(rows_v.at[s], self2_hbm.at[pl.ds(lo, CH)])
            pltpu.sync_copy(rows_v.at[s + 1], dinve_hbm.at[pl.ds(lo, CH)])

    pltpu.sync_copy(src_hbm.at[wid], src_v)
    pltpu.sync_copy(dst_hbm.at[wid], dst_v)
    plsc.subcore_barrier()
    _agg_pipeline(src_v, dst_v, rows_v, acc_sh, ytab_sh, gsem, ssem)
    plsc.subcore_barrier()
    _writeout(acc_sh, out_hbm, cid, sid)


def _tc_mm1_body(x_ref, w1_ref, o_ref):
    o_ref[...] = jnp.dot(x_ref[...], w1_ref[...],
                         preferred_element_type=_f32)


_tc_mm1 = pl.pallas_call(
    _tc_mm1_body,
    out_shape=jax.ShapeDtypeStruct((N_NODES, H1), _f32),
)


def _tc_out_body(accp_ref, self2_ref, dinvp_ref, w24_ref, b2t_ref, out_ref):
    a = accp_ref[0, :NP4, :] + accp_ref[1, :NP4, :]
    zp = a * dinvp_ref[:NP4, :] + self2_ref[:NP4, :]
    out_ref[...] = (
        jnp.dot(zp, w24_ref[...], preferred_element_type=_f32) + b2t_ref[...]
    )


_tc_out = pl.pallas_call(
    _tc_out_body,
    out_shape=jax.ShapeDtypeStruct((NP4, 4 * H2), _f32),
)


def kernel(x, edge_index, W1, b1, W2, b2):
    ei = edge_index.astype(jnp.int32)
    pad = jnp.broadcast_to(
        jnp.array([[0], [DUMMY]], jnp.int32), (2, E_PAD - N_EDGES)
    )
    ei_p = jnp.concatenate([ei, pad], axis=1)
    src_p = ei_p[0].reshape(NW, NCH, CH)
    dst_p = ei_p[1].reshape(NW, NCH, CH)

    degp = _deg_kernel(dst_p)
    xw = _tc_mm1(x, W1)
    xw_p = jnp.pad(xw, ((0, ACC_R - N_NODES), (0, 0)))

    acc1 = _agg1_kernel(xw_p, degp, src_p, dst_p)
    acc2, self2, dinve = _agg2_kernel(acc1, xw_p, degp, b1, src_p, dst_p)

    w24 = jax.scipy.linalg.block_diag(W2, W2, W2, W2)
    b2t = jnp.tile(b2, 4).reshape(1, 4 * H2)
    out4 = _tc_out(
        acc2.reshape(NC, AP4, 4 * H1),
        self2.reshape(AP4, 4 * H1),
        dinve.reshape(AP4, 4 * H1),
        w24,
        b2t,
    )
    return out4.reshape(N_NODES, H2)
